# baseline (speedup 1.0000x reference)
import functools

import jax
import jax.numpy as jnp
from jax import lax
from jax.experimental import pallas as pl
from jax.experimental.pallas import tpu as pltpu
from jax.experimental.pallas import tpu_sc as plsc

_L = 16
_NC = 2
_NS = 16
_NW = _NC * _NS


@functools.lru_cache(maxsize=None)
def _build(T, B):
    b_per_w = B // _NW
    mesh = plsc.VectorSubcoreMesh(core_axis_name="c", subcore_axis_name="s")

    @functools.partial(
        pl.kernel,
        mesh=mesh,
        out_type=jax.ShapeDtypeStruct((B,), jnp.float32),
        scratch_types=[
            pltpu.VMEM((b_per_w,), jnp.int32),
            pltpu.VMEM((b_per_w,), jnp.float32),
            pltpu.SemaphoreType.DMA,
        ],
        compiler_params=pltpu.CompilerParams(needs_layout_passes=False),
    )
    def gather_kernel(arr_hbm, t_hbm, out_hbm, idx_v, val_v, sem):
        wid = lax.axis_index("s") * _NC + lax.axis_index("c")
        base = wid * b_per_w
        pltpu.sync_copy(t_hbm.at[pl.ds(base, b_per_w)], idx_v)

        chunk = 128
        cps = [
            pltpu.async_copy(
                arr_hbm.at[idx_v.at[pl.ds(j * chunk, chunk)]],
                val_v.at[pl.ds(j * chunk, chunk)],
                sem,
            )
            for j in range(b_per_w // chunk)
        ]
        for cp in cps:
            cp.wait()

        pltpu.sync_copy(val_v, out_hbm.at[pl.ds(base, b_per_w)])

    return gather_kernel


def kernel(arr, t, x):
    B = t.shape[0]
    out = _build(arr.shape[0], B)(arr, t)
    return out.reshape((B,) + (1,) * (x.ndim - 1))


# speedup vs baseline: 1.4267x; 1.4267x over previous
import functools

import jax
import jax.numpy as jnp
from jax import lax
from jax.experimental import pallas as pl
from jax.experimental.pallas import tpu as pltpu
from jax.experimental.pallas import tpu_sc as plsc

_L = 16
_NC = 2
_NS = 16
_NW = _NC * _NS


@functools.lru_cache(maxsize=None)
def _build(T, B):
    b_per_w = B // _NW
    mesh = plsc.VectorSubcoreMesh(core_axis_name="c", subcore_axis_name="s")

    @functools.partial(
        pl.kernel,
        mesh=mesh,
        out_type=jax.ShapeDtypeStruct((B,), jnp.float32),
        scratch_types=[
            pltpu.VMEM((T,), jnp.float32),
            pltpu.VMEM((b_per_w,), jnp.int32),
            pltpu.VMEM((b_per_w,), jnp.float32),
            pltpu.SemaphoreType.DMA,
            pltpu.SemaphoreType.DMA,
        ],
        compiler_params=pltpu.CompilerParams(needs_layout_passes=False),
    )
    def gather_kernel(arr_hbm, t_hbm, out_hbm, tab_v, idx_v, val_v, sem_a, sem_b):
        wid = lax.axis_index("s") * _NC + lax.axis_index("c")
        base = wid * b_per_w
        tab_cp = pltpu.async_copy(arr_hbm, tab_v, sem_a)
        idx_cp = pltpu.async_copy(t_hbm.at[pl.ds(base, b_per_w)], idx_v, sem_b)
        tab_cp.wait()
        idx_cp.wait()

        half = b_per_w // 2
        for i in range(half // _L):
            off = i * _L
            idx = idx_v[pl.ds(off, _L)]
            val_v[pl.ds(off, _L)] = plsc.load_gather(tab_v, [idx])
        out1 = pltpu.async_copy(
            val_v.at[pl.ds(0, half)], out_hbm.at[pl.ds(base, half)], sem_a
        )
        for i in range(half // _L, b_per_w // _L):
            off = i * _L
            idx = idx_v[pl.ds(off, _L)]
            val_v[pl.ds(off, _L)] = plsc.load_gather(tab_v, [idx])
        out2 = pltpu.async_copy(
            val_v.at[pl.ds(half, half)], out_hbm.at[pl.ds(base + half, half)], sem_b
        )
        out1.wait()
        out2.wait()

    return gather_kernel


def kernel(arr, t, x):
    B = t.shape[0]
    out = _build(arr.shape[0], B)(arr, t)
    return out.reshape((B,) + (1,) * (x.ndim - 1))
